# Initial kernel scaffold; baseline (speedup 1.0000x reference)
import functools

import jax
import jax.numpy as jnp
from jax import lax
from jax.experimental import pallas as pl
from jax.experimental.pallas import tpu as pltpu
from jax.experimental.pallas import tpu_sc as plsc

N = 100000
E = 6400000
NP = 102400
DUMP = 100352
DELTA_X = 0.1
DELTA_Y = 0.1
F_CONST = 1.0

NC = 2
NS = 16
CHUNK = 2048
NCHUNKS = E // CHUNK
ROWS_EI = 2 * E // 128
ACCW = 12 * NP
ZW = ACCW // NS
ZB = 7680

_mesh = plsc.VectorSubcoreMesh(core_axis_name="c", subcore_axis_name="s")


def _sc_body(ei2, ea2, xop, xap, out,
             srcb, dstb, eab, xab, xosb, xodb, sidx, sval, zbuf,
             acc, xo_sh, xa_sh):
    cid = lax.axis_index("c")
    sid = lax.axis_index("s")
    wid = cid * NS + sid

    iota = lax.broadcasted_iota(jnp.int32, (16,), 0)
    zeros16 = jnp.zeros((16,), jnp.float32)
    ones16 = jnp.ones((16,), jnp.float32)

    toff = sid * (NP // NS)
    pltpu.sync_copy(xop.at[pl.ds(toff, NP // NS)], xo_sh.at[pl.ds(toff, NP // NS)])
    pltpu.sync_copy(xap.at[pl.ds(toff, NP // NS)], xa_sh.at[pl.ds(toff, NP // NS)])

    def zb_body(i, _):
        zbuf[pl.ds(i * 16, 16)] = zeros16
        return 0
    lax.fori_loop(0, ZB // 16, zb_body, 0)

    def zc_body(i, _):
        pltpu.sync_copy(zbuf, acc.at[pl.ds(sid * ZW + i * ZB, ZB)])
        return 0
    lax.fori_loop(0, ZW // ZB, zc_body, 0)

    def pf_body(i, _):
        r = i // 16
        l = (i % 16) * 16
        sval[32 + r, pl.ds(l, 16)] = ones16
        sval[80 + r, pl.ds(l, 16)] = ones16
        return 0
    lax.fori_loop(0, 16 * 8, pf_body, 0)

    plsc.subcore_barrier()

    nch = jnp.where(wid < NCHUNKS % 32, NCHUNKS // 32 + 1, NCHUNKS // 32)
    base = wid * (NCHUNKS // 32) + jnp.minimum(wid, NCHUNKS % 32)

    def chunk_body(i, _):
        ch = base + i
        pltpu.sync_copy(ei2.at[pl.ds(16 * ch, 16)], srcb)
        pltpu.sync_copy(ei2.at[pl.ds(ROWS_EI // 2 + 16 * ch, 16)], dstb)
        pltpu.sync_copy(ea2.at[pl.ds(32 * ch, 32)], eab)
        pltpu.sync_copy(xa_sh.at[srcb], xab)
        pltpu.sync_copy(xo_sh.at[srcb], xosb)
        pltpu.sync_copy(xo_sh.at[dstb], xodb)

        def step(t, _):
            g = t // 8
            l = (t % 8) * 16
            dst16 = dstb[g, pl.ds(l, 16)]
            xa16 = xab[g, pl.ds(l, 16)]
            du = xodb[g, pl.ds(l, 16)] - xosb[g, pl.ds(l, 16)]
            er = t // 4
            eo = (t % 4) * 32
            lx = iota * 2 + eo
            eax = plsc.load_gather(eab, [jnp.full((16,), er, jnp.int32), lx])
            eay = plsc.load_gather(eab, [jnp.full((16,), er, jnp.int32), lx + 1])

            def onedir(ea, b_sa, b_su, b_c):
                neg = ea < 0.0
                m = ea != 0.0
                b0 = dst16 + jnp.where(neg, NP, 0)
                bs = jnp.where(m, b0, DUMP)
                return bs + b_sa, bs + b_su, bs + b_c, du / ea

            ix_sa, ix_su, ix_c, vx = onedir(eax, 0, 2 * NP, 4 * NP)
            iy_sa, iy_su, iy_c, vy = onedir(eay, 6 * NP, 8 * NP, 10 * NP)

            sidx[g, pl.ds(l, 16)] = ix_sa
            sidx[16 + g, pl.ds(l, 16)] = ix_su
            sidx[32 + g, pl.ds(l, 16)] = ix_c
            sidx[48 + g, pl.ds(l, 16)] = iy_sa
            sidx[64 + g, pl.ds(l, 16)] = iy_su
            sidx[80 + g, pl.ds(l, 16)] = iy_c
            sval[g, pl.ds(l, 16)] = xa16
            sval[16 + g, pl.ds(l, 16)] = vx
            sval[48 + g, pl.ds(l, 16)] = xa16
            sval[64 + g, pl.ds(l, 16)] = vy
            return 0

        lax.fori_loop(0, CHUNK // 16, step, 0)
        pltpu.sync_copy(sval, acc.at[sidx], add=True)
        return 0

    lax.fori_loop(0, nch, chunk_body, 0)

    plsc.subcore_barrier()

    pltpu.sync_copy(acc.at[pl.ds(sid * ZW, ZW)], out.at[cid, pl.ds(sid * ZW, ZW)])


@functools.partial(
    pl.kernel,
    out_type=jax.ShapeDtypeStruct((NC, ACCW), jnp.float32),
    mesh=_mesh,
    scratch_types=[
        pltpu.VMEM((16, 128), jnp.int32),
        pltpu.VMEM((16, 128), jnp.int32),
        pltpu.VMEM((32, 128), jnp.float32),
        pltpu.VMEM((16, 128), jnp.float32),
        pltpu.VMEM((16, 128), jnp.float32),
        pltpu.VMEM((16, 128), jnp.float32),
        pltpu.VMEM((96, 128), jnp.int32),
        pltpu.VMEM((96, 128), jnp.float32),
        pltpu.VMEM((ZB,), jnp.float32),
        pltpu.VMEM_SHARED((ACCW,), jnp.float32),
        pltpu.VMEM_SHARED((NP,), jnp.float32),
        pltpu.VMEM_SHARED((NP,), jnp.float32),
    ],
)
def _sc_scatter(ei2, ea2, xop, xap, out, *scratch):
    _sc_body(ei2, ea2, xop, xap, out, *scratch)


_KC = 12800


def _combine_body(a_ref, o_ref):
    a = a_ref[...]
    s = a[0:12] + a[12:24]
    mcxp = jnp.maximum(s[4:5], 1.0)
    mcxm = jnp.maximum(s[5:6], 1.0)
    mcyp = jnp.maximum(s[10:11], 1.0)
    mcym = jnp.maximum(s[11:12], 1.0)
    axp = s[0:1] / mcxp
    axm = s[1:2] / mcxm
    uxp = s[2:3] / mcxp
    uxm = s[3:4] / mcxm
    ayp = s[6:7] / mcyp
    aym = s[7:8] / mcym
    uyp = s[8:9] / mcyp
    uym = s[9:10] / mcym
    loss = (axp * uxp - axm * uxm) / DELTA_X \
         + (ayp * uyp - aym * uym) / DELTA_Y + F_CONST
    o_ref[...] = loss


_combine = pl.pallas_call(
    _combine_body,
    grid=(NP // _KC,),
    in_specs=[pl.BlockSpec((24, _KC), lambda i: (0, i))],
    out_specs=pl.BlockSpec((1, _KC), lambda i: (0, i)),
    out_shape=jax.ShapeDtypeStruct((1, NP), jnp.float32),
)


@jax.jit
def kernel(x_out, x_a, edge_attr, edge_index):
    ei2 = edge_index.reshape(ROWS_EI, 128)
    ea2 = edge_attr.reshape(ROWS_EI, 128)
    xop = jnp.pad(x_out[:, 0], (0, NP - N))
    xap = jnp.pad(x_a[:, 0], (0, NP - N))
    acc = _sc_scatter(ei2, ea2, xop, xap)
    loss = _combine(acc.reshape(24, NP))
    return loss.reshape(NP, 1)[:N]


# speedup vs baseline: 27.5081x; 27.5081x over previous
import functools

import jax
import jax.numpy as jnp
from jax import lax
from jax.experimental import pallas as pl
from jax.experimental.pallas import tpu as pltpu
from jax.experimental.pallas import tpu_sc as plsc

N = 100000
E = 6400000
NP = 100480
DUMP = 100224
DELTA_X = 0.1
DELTA_Y = 0.1
F_CONST = 1.0

NC = 2
NS = 16
CHUNK = 2048
NCHUNKS = E // CHUNK
ACCW = 12 * NP
ZW = 76800
ZWL = ACCW - 15 * ZW
ZB = 7680
TS = 6400
TSL = NP - 15 * TS

_mesh = plsc.VectorSubcoreMesh(core_axis_name="c", subcore_axis_name="s")

_GDN = lax.GatherDimensionNumbers(
    offset_dims=(), collapsed_slice_dims=(0,), start_index_map=(0,))


def _dyn_gather(v, idx):
    return lax.gather(v, idx[:, None], _GDN, slice_sizes=(1,),
                      mode=lax.GatherScatterMode.PROMISE_IN_BOUNDS)


def _sc_body(ei2, ea2, xop, xap, out,
             srcb, dstb, eab, xab, xosb, xodb, sidx, sval, zbuf,
             acc, xo_sh):
    cid = lax.axis_index("c")
    sid = lax.axis_index("s")
    wid = cid * NS + sid

    iota = lax.broadcasted_iota(jnp.int32, (16,), 0)
    zeros16 = jnp.zeros((16,), jnp.float32)
    ones16 = jnp.ones((16,), jnp.float32)

    toff = sid * TS

    @pl.when(sid < 15)
    def _():
        pltpu.sync_copy(xop.at[pl.ds(toff, TS)], xo_sh.at[pl.ds(toff, TS)])

    @pl.when(sid == 15)
    def _():
        pltpu.sync_copy(xop.at[pl.ds(15 * TS, TSL)], xo_sh.at[pl.ds(15 * TS, TSL)])

    def zb_body(i, _):
        zbuf[pl.ds(i * 16, 16)] = zeros16
        return 0
    lax.fori_loop(0, ZB // 16, zb_body, 0)

    def zc_body(i, _):
        pltpu.sync_copy(zbuf, acc.at[pl.ds(sid * ZW + i * ZB, ZB)])
        return 0
    ncp = jnp.where(sid < 15, ZW // ZB, ZWL // ZB)
    lax.fori_loop(0, ncp, zc_body, 0)

    def pf_body(i, _):
        sval[pl.ds(2 * CHUNK + i * 16, 16)] = ones16
        sval[pl.ds(5 * CHUNK + i * 16, 16)] = ones16
        return 0
    lax.fori_loop(0, CHUNK // 16, pf_body, 0)

    plsc.subcore_barrier()

    nch = jnp.where(wid < NCHUNKS % 32, NCHUNKS // 32 + 1, NCHUNKS // 32)
    base = wid * (NCHUNKS // 32) + jnp.minimum(wid, NCHUNKS % 32)

    def chunk_body(i, _):
        ch = base + i
        pltpu.sync_copy(ei2.at[pl.ds(ch * CHUNK, CHUNK)], srcb)
        pltpu.sync_copy(ei2.at[pl.ds(E + ch * CHUNK, CHUNK)], dstb)
        pltpu.sync_copy(ea2.at[pl.ds(ch * 2 * CHUNK, 2 * CHUNK)], eab)
        pltpu.sync_copy(xap.at[srcb], xab)
        pltpu.sync_copy(xo_sh.at[srcb], xosb)
        pltpu.sync_copy(xo_sh.at[dstb], xodb)

        def step(t, _):
            l = t * 16
            dst16 = dstb[pl.ds(l, 16)]
            xa16 = xab[pl.ds(l, 16)]
            du = xodb[pl.ds(l, 16)] - xosb[pl.ds(l, 16)]
            v0 = eab[pl.ds(t * 32, 16)]
            v1 = eab[pl.ds(t * 32 + 16, 16)]
            idx_e = (iota * 2) % 16
            lo = iota < 8
            eax = jnp.where(lo, _dyn_gather(v0, idx_e), _dyn_gather(v1, idx_e))
            idx_o = (iota * 2 + 1) % 16
            eay = jnp.where(lo, _dyn_gather(v0, idx_o), _dyn_gather(v1, idx_o))

            def onedir(ea, b_sa, b_su, b_c):
                neg = ea < 0.0
                m = ea != 0.0
                b0 = dst16 + jnp.where(neg, NP, 0)
                bs = jnp.where(m, b0, DUMP)
                return bs + b_sa, bs + b_su, bs + b_c, du / ea

            ix_sa, ix_su, ix_c, vx = onedir(eax, 0, 2 * NP, 4 * NP)
            iy_sa, iy_su, iy_c, vy = onedir(eay, 6 * NP, 8 * NP, 10 * NP)

            sidx[pl.ds(l, 16)] = ix_sa
            sidx[pl.ds(CHUNK + l, 16)] = ix_su
            sidx[pl.ds(2 * CHUNK + l, 16)] = ix_c
            sidx[pl.ds(3 * CHUNK + l, 16)] = iy_sa
            sidx[pl.ds(4 * CHUNK + l, 16)] = iy_su
            sidx[pl.ds(5 * CHUNK + l, 16)] = iy_c
            sval[pl.ds(l, 16)] = xa16
            sval[pl.ds(CHUNK + l, 16)] = vx
            sval[pl.ds(3 * CHUNK + l, 16)] = xa16
            sval[pl.ds(4 * CHUNK + l, 16)] = vy
            return 0

        lax.fori_loop(0, CHUNK // 16, step, 0)
        pltpu.sync_copy(sval, acc.at[sidx], add=True)
        return 0

    lax.fori_loop(0, nch, chunk_body, 0)

    plsc.subcore_barrier()

    obase = cid * ACCW + sid * ZW

    @pl.when(sid < 15)
    def _():
        pltpu.sync_copy(acc.at[pl.ds(sid * ZW, ZW)], out.at[pl.ds(obase, ZW)])

    @pl.when(sid == 15)
    def _():
        pltpu.sync_copy(acc.at[pl.ds(15 * ZW, ZWL)],
                        out.at[pl.ds(cid * ACCW + 15 * ZW, ZWL)])


@functools.partial(
    pl.kernel,
    out_type=jax.ShapeDtypeStruct((NC * ACCW,), jnp.float32),
    mesh=_mesh,
    scratch_types=[
        pltpu.VMEM((CHUNK,), jnp.int32),
        pltpu.VMEM((CHUNK,), jnp.int32),
        pltpu.VMEM((2 * CHUNK,), jnp.float32),
        pltpu.VMEM((CHUNK,), jnp.float32),
        pltpu.VMEM((CHUNK,), jnp.float32),
        pltpu.VMEM((CHUNK,), jnp.float32),
        pltpu.VMEM((6 * CHUNK,), jnp.int32),
        pltpu.VMEM((6 * CHUNK,), jnp.float32),
        pltpu.VMEM((ZB,), jnp.float32),
        pltpu.VMEM_SHARED((ACCW,), jnp.float32),
        pltpu.VMEM_SHARED((NP,), jnp.float32),
    ],
)
def _sc_scatter(ei2, ea2, xop, xap, out, *scratch):
    _sc_body(ei2, ea2, xop, xap, out, *scratch)


_KC = 20096


def _combine_body(a_ref, o_ref):
    a = a_ref[...]
    s = a[0:12] + a[12:24]
    mcxp = jnp.maximum(s[4:5], 1.0)
    mcxm = jnp.maximum(s[5:6], 1.0)
    mcyp = jnp.maximum(s[10:11], 1.0)
    mcym = jnp.maximum(s[11:12], 1.0)
    axp = s[0:1] / mcxp
    axm = s[1:2] / mcxm
    uxp = s[2:3] / mcxp
    uxm = s[3:4] / mcxm
    ayp = s[6:7] / mcyp
    aym = s[7:8] / mcym
    uyp = s[8:9] / mcyp
    uym = s[9:10] / mcym
    loss = (axp * uxp - axm * uxm) / DELTA_X \
         + (ayp * uyp - aym * uym) / DELTA_Y + F_CONST
    o_ref[...] = loss


_combine = pl.pallas_call(
    _combine_body,
    grid=(NP // _KC,),
    in_specs=[pl.BlockSpec((24, _KC), lambda i: (0, i))],
    out_specs=pl.BlockSpec((1, _KC), lambda i: (0, i)),
    out_shape=jax.ShapeDtypeStruct((1, NP), jnp.float32),
)


@jax.jit
def kernel(x_out, x_a, edge_attr, edge_index):
    ei2 = edge_index.reshape(2 * E)
    ea2 = edge_attr.reshape(2 * E)
    xop = jnp.pad(x_out[:, 0], (0, NP - N))
    xap = x_a[:, 0]
    acc = _sc_scatter(ei2, ea2, xop, xap)
    loss = _combine(acc.reshape(24, NP))
    return loss.reshape(NP, 1)[:N]


# speedup vs baseline: 197.4644x; 7.1784x over previous
import functools

import jax
import jax.numpy as jnp
from jax import lax
from jax.experimental import pallas as pl
from jax.experimental.pallas import tpu as pltpu
from jax.experimental.pallas import tpu_sc as plsc

N = 100000
E = 6400000
NP = 100480
DUMP = 100224
DELTA_X = 0.1
DELTA_Y = 0.1
F_CONST = 1.0

NC = 2
NS = 16
CHUNK = 2048
NCHUNKS = E // CHUNK
ACCW = 12 * NP
ZW = 76800
ZWL = ACCW - 15 * ZW
ZB = 7680
TS = 6400
TSL = NP - 15 * TS

_mesh = plsc.VectorSubcoreMesh(core_axis_name="c", subcore_axis_name="s")

_GDN = lax.GatherDimensionNumbers(
    offset_dims=(), collapsed_slice_dims=(0,), start_index_map=(0,))


def _dyn_gather(v, idx):
    return lax.gather(v, idx[:, None], _GDN, slice_sizes=(1,),
                      mode=lax.GatherScatterMode.PROMISE_IN_BOUNDS)


def _sc_body(ei, eax_h, eay_h, xop, xap, out,
             srcb, dstb, eaxb, eayb, xab, xosb, xodb, sidx, sval, zbuf,
             acc, xo_sh):
    cid = lax.axis_index("c")
    sid = lax.axis_index("s")
    wid = cid * NS + sid

    iota = lax.broadcasted_iota(jnp.int32, (16,), 0)
    zeros16 = jnp.zeros((16,), jnp.float32)
    ones16 = jnp.ones((16,), jnp.float32)

    toff = sid * TS

    @pl.when(sid < 15)
    def _():
        pltpu.sync_copy(xop.at[pl.ds(toff, TS)], xo_sh.at[pl.ds(toff, TS)])

    @pl.when(sid == 15)
    def _():
        pltpu.sync_copy(xop.at[pl.ds(15 * TS, TSL)], xo_sh.at[pl.ds(15 * TS, TSL)])

    def zb_body(i, _):
        zbuf[pl.ds(i * 16, 16)] = zeros16
        return 0
    lax.fori_loop(0, ZB // 16, zb_body, 0)

    def zc_body(i, _):
        pltpu.sync_copy(zbuf, acc.at[pl.ds(sid * ZW + i * ZB, ZB)])
        return 0
    ncp = jnp.where(sid < 15, ZW // ZB, ZWL // ZB)
    lax.fori_loop(0, ncp, zc_body, 0)

    def pf_body(i, _):
        sval[pl.ds(2 * CHUNK + i * 16, 16)] = ones16
        sval[pl.ds(5 * CHUNK + i * 16, 16)] = ones16
        return 0
    lax.fori_loop(0, CHUNK // 16, pf_body, 0)

    plsc.subcore_barrier()

    nch = jnp.where(wid < NCHUNKS % 32, NCHUNKS // 32 + 1, NCHUNKS // 32)
    base = wid * (NCHUNKS // 32) + jnp.minimum(wid, NCHUNKS % 32)

    def chunk_body(i, _):
        ch = base + i
        pltpu.sync_copy(ei.at[0, pl.ds(ch * CHUNK, CHUNK)], srcb)
        pltpu.sync_copy(ei.at[1, pl.ds(ch * CHUNK, CHUNK)], dstb)
        pltpu.sync_copy(eax_h.at[pl.ds(ch * CHUNK, CHUNK)], eaxb)
        pltpu.sync_copy(eay_h.at[pl.ds(ch * CHUNK, CHUNK)], eayb)
        pltpu.sync_copy(xap.at[srcb], xab)
        pltpu.sync_copy(xo_sh.at[srcb], xosb)
        pltpu.sync_copy(xo_sh.at[dstb], xodb)

        def step(t, _):
            l = t * 16
            dst16 = dstb[pl.ds(l, 16)]
            xa16 = xab[pl.ds(l, 16)]
            du = xodb[pl.ds(l, 16)] - xosb[pl.ds(l, 16)]
            eax = eaxb[pl.ds(l, 16)]
            eay = eayb[pl.ds(l, 16)]

            def onedir(ea, b_sa, b_su, b_c):
                neg = ea < 0.0
                m = ea != 0.0
                b0 = dst16 + jnp.where(neg, NP, 0)
                bs = jnp.where(m, b0, DUMP)
                return bs + b_sa, bs + b_su, bs + b_c, du / ea

            ix_sa, ix_su, ix_c, vx = onedir(eax, 0, 2 * NP, 4 * NP)
            iy_sa, iy_su, iy_c, vy = onedir(eay, 6 * NP, 8 * NP, 10 * NP)

            sidx[pl.ds(l, 16)] = ix_sa
            sidx[pl.ds(CHUNK + l, 16)] = ix_su
            sidx[pl.ds(2 * CHUNK + l, 16)] = ix_c
            sidx[pl.ds(3 * CHUNK + l, 16)] = iy_sa
            sidx[pl.ds(4 * CHUNK + l, 16)] = iy_su
            sidx[pl.ds(5 * CHUNK + l, 16)] = iy_c
            sval[pl.ds(l, 16)] = xa16
            sval[pl.ds(CHUNK + l, 16)] = vx
            sval[pl.ds(3 * CHUNK + l, 16)] = xa16
            sval[pl.ds(4 * CHUNK + l, 16)] = vy
            return 0

        lax.fori_loop(0, CHUNK // 16, step, 0)
        pltpu.sync_copy(sval, acc.at[sidx], add=True)
        return 0

    lax.fori_loop(0, nch, chunk_body, 0)

    plsc.subcore_barrier()

    obase = cid * ACCW + sid * ZW

    @pl.when(sid < 15)
    def _():
        pltpu.sync_copy(acc.at[pl.ds(sid * ZW, ZW)], out.at[pl.ds(obase, ZW)])

    @pl.when(sid == 15)
    def _():
        pltpu.sync_copy(acc.at[pl.ds(15 * ZW, ZWL)],
                        out.at[pl.ds(cid * ACCW + 15 * ZW, ZWL)])


@functools.partial(
    pl.kernel,
    out_type=jax.ShapeDtypeStruct((NC * ACCW,), jnp.float32),
    mesh=_mesh,
    scratch_types=[
        pltpu.VMEM((CHUNK,), jnp.int32),
        pltpu.VMEM((CHUNK,), jnp.int32),
        pltpu.VMEM((CHUNK,), jnp.float32),
        pltpu.VMEM((CHUNK,), jnp.float32),
        pltpu.VMEM((CHUNK,), jnp.float32),
        pltpu.VMEM((CHUNK,), jnp.float32),
        pltpu.VMEM((CHUNK,), jnp.float32),
        pltpu.VMEM((6 * CHUNK,), jnp.int32),
        pltpu.VMEM((6 * CHUNK,), jnp.float32),
        pltpu.VMEM((ZB,), jnp.float32),
        pltpu.VMEM_SHARED((ACCW,), jnp.float32),
        pltpu.VMEM_SHARED((NP,), jnp.float32),
    ],
)
def _sc_scatter(ei, eax_h, eay_h, xop, xap, out, *scratch):
    _sc_body(ei, eax_h, eay_h, xop, xap, out, *scratch)


_KC = 20096


def _combine_body(a_ref, o_ref):
    a = a_ref[...]
    s = a[0:12] + a[12:24]
    mcxp = jnp.maximum(s[4:5], 1.0)
    mcxm = jnp.maximum(s[5:6], 1.0)
    mcyp = jnp.maximum(s[10:11], 1.0)
    mcym = jnp.maximum(s[11:12], 1.0)
    axp = s[0:1] / mcxp
    axm = s[1:2] / mcxm
    uxp = s[2:3] / mcxp
    uxm = s[3:4] / mcxm
    ayp = s[6:7] / mcyp
    aym = s[7:8] / mcym
    uyp = s[8:9] / mcyp
    uym = s[9:10] / mcym
    loss = (axp * uxp - axm * uxm) / DELTA_X \
         + (ayp * uyp - aym * uym) / DELTA_Y + F_CONST
    o_ref[...] = loss


_combine = pl.pallas_call(
    _combine_body,
    grid=(NP // _KC,),
    in_specs=[pl.BlockSpec((24, _KC), lambda i: (0, i))],
    out_specs=pl.BlockSpec((1, _KC), lambda i: (0, i)),
    out_shape=jax.ShapeDtypeStruct((1, NP), jnp.float32),
)


@jax.jit
def kernel(x_out, x_a, edge_attr, edge_index):
    xop = jnp.pad(x_out[:, 0], (0, NP - N))
    xap = x_a[:, 0]
    eax_h = edge_attr[:, 0]
    eay_h = edge_attr[:, 1]
    acc = _sc_scatter(edge_index, eax_h, eay_h, xop, xap)
    loss = _combine(acc.reshape(24, NP))
    return loss.reshape(NP, 1)[:N]


# speedup vs baseline: 238.4122x; 1.2074x over previous
import functools

import jax
import jax.numpy as jnp
from jax import lax
from jax.experimental import pallas as pl
from jax.experimental.pallas import tpu as pltpu
from jax.experimental.pallas import tpu_sc as plsc

N = 100000
E = 6400000
NP = 100480
DUMP = 100224
DELTA_X = 0.1
DELTA_Y = 0.1
F_CONST = 1.0

NC = 2
NS = 16
CHUNK = 2048
NCHUNKS = E // CHUNK
ACCW = 12 * NP
ZW = 76800
ZWL = ACCW - 15 * ZW
ZB = 7680
TS = 6400
TSL = NP - 15 * TS

_mesh = plsc.VectorSubcoreMesh(core_axis_name="c", subcore_axis_name="s")

_GDN = lax.GatherDimensionNumbers(
    offset_dims=(), collapsed_slice_dims=(0,), start_index_map=(0,))


def _dyn_gather(v, idx):
    return lax.gather(v, idx[:, None], _GDN, slice_sizes=(1,),
                      mode=lax.GatherScatterMode.PROMISE_IN_BOUNDS)


def _sc_body(ei, eax_h, eay_h, xop, xap, out,
             srcb, dstb, eaxb, eayb, xab, xosb, xodb, sidx, sval, zbuf,
             acc, xo_sh, sem):
    cid = lax.axis_index("c")
    sid = lax.axis_index("s")
    wid = cid * NS + sid

    iota = lax.broadcasted_iota(jnp.int32, (16,), 0)
    zeros16 = jnp.zeros((16,), jnp.float32)
    ones16 = jnp.ones((16,), jnp.float32)

    toff = sid * TS

    @pl.when(sid < 15)
    def _():
        pltpu.sync_copy(xop.at[pl.ds(toff, TS)], xo_sh.at[pl.ds(toff, TS)])

    @pl.when(sid == 15)
    def _():
        pltpu.sync_copy(xop.at[pl.ds(15 * TS, TSL)], xo_sh.at[pl.ds(15 * TS, TSL)])

    def zb_body(i, _):
        zbuf[pl.ds(i * 16, 16)] = zeros16
        return 0
    lax.fori_loop(0, ZB // 16, zb_body, 0)

    def zc_body(i, _):
        pltpu.sync_copy(zbuf, acc.at[pl.ds(sid * ZW + i * ZB, ZB)])
        return 0
    ncp = jnp.where(sid < 15, ZW // ZB, ZWL // ZB)
    lax.fori_loop(0, ncp, zc_body, 0)

    def pf_body(i, _):
        sval[pl.ds(2 * CHUNK + i * 16, 16)] = ones16
        sval[pl.ds(5 * CHUNK + i * 16, 16)] = ones16
        return 0
    lax.fori_loop(0, CHUNK // 16, pf_body, 0)

    plsc.subcore_barrier()

    nch = jnp.where(wid < NCHUNKS % 32, NCHUNKS // 32 + 1, NCHUNKS // 32)
    base = wid * (NCHUNKS // 32) + jnp.minimum(wid, NCHUNKS % 32)

    def chunk_body(i, _):
        ch = base + i
        h1 = pltpu.async_copy(ei.at[0, pl.ds(ch * CHUNK, CHUNK)], srcb, sem)
        h2 = pltpu.async_copy(ei.at[1, pl.ds(ch * CHUNK, CHUNK)], dstb, sem)
        h3 = pltpu.async_copy(eax_h.at[pl.ds(ch * CHUNK, CHUNK)], eaxb, sem)
        h4 = pltpu.async_copy(eay_h.at[pl.ds(ch * CHUNK, CHUNK)], eayb, sem)
        h1.wait(); h2.wait(); h3.wait(); h4.wait()
        pltpu.sync_copy(xap.at[srcb], xab)
        pltpu.sync_copy(xo_sh.at[srcb], xosb)
        pltpu.sync_copy(xo_sh.at[dstb], xodb)

        def step(t, _):
            l = t * 16
            dst16 = dstb[pl.ds(l, 16)]
            xa16 = xab[pl.ds(l, 16)]
            du = xodb[pl.ds(l, 16)] - xosb[pl.ds(l, 16)]
            eax = eaxb[pl.ds(l, 16)]
            eay = eayb[pl.ds(l, 16)]

            def onedir(ea, b_sa, b_su, b_c):
                neg = ea < 0.0
                m = ea != 0.0
                b0 = dst16 + jnp.where(neg, NP, 0)
                bs = jnp.where(m, b0, DUMP)
                return bs + b_sa, bs + b_su, bs + b_c, du / ea

            ix_sa, ix_su, ix_c, vx = onedir(eax, 0, 2 * NP, 4 * NP)
            iy_sa, iy_su, iy_c, vy = onedir(eay, 6 * NP, 8 * NP, 10 * NP)

            sidx[pl.ds(l, 16)] = ix_sa
            sidx[pl.ds(CHUNK + l, 16)] = ix_su
            sidx[pl.ds(2 * CHUNK + l, 16)] = ix_c
            sidx[pl.ds(3 * CHUNK + l, 16)] = iy_sa
            sidx[pl.ds(4 * CHUNK + l, 16)] = iy_su
            sidx[pl.ds(5 * CHUNK + l, 16)] = iy_c
            sval[pl.ds(l, 16)] = xa16
            sval[pl.ds(CHUNK + l, 16)] = vx
            sval[pl.ds(3 * CHUNK + l, 16)] = xa16
            sval[pl.ds(4 * CHUNK + l, 16)] = vy
            return 0

        lax.fori_loop(0, CHUNK // 16, step, 0)
        pltpu.sync_copy(sval, acc.at[sidx], add=True)
        return 0

    lax.fori_loop(0, nch, chunk_body, 0)

    plsc.subcore_barrier()

    obase = cid * ACCW + sid * ZW

    @pl.when(sid < 15)
    def _():
        pltpu.sync_copy(acc.at[pl.ds(sid * ZW, ZW)], out.at[pl.ds(obase, ZW)])

    @pl.when(sid == 15)
    def _():
        pltpu.sync_copy(acc.at[pl.ds(15 * ZW, ZWL)],
                        out.at[pl.ds(cid * ACCW + 15 * ZW, ZWL)])


@functools.partial(
    pl.kernel,
    out_type=jax.ShapeDtypeStruct((NC * ACCW,), jnp.float32),
    mesh=_mesh,
    scratch_types=[
        pltpu.VMEM((CHUNK,), jnp.int32),
        pltpu.VMEM((CHUNK,), jnp.int32),
        pltpu.VMEM((CHUNK,), jnp.float32),
        pltpu.VMEM((CHUNK,), jnp.float32),
        pltpu.VMEM((CHUNK,), jnp.float32),
        pltpu.VMEM((CHUNK,), jnp.float32),
        pltpu.VMEM((CHUNK,), jnp.float32),
        pltpu.VMEM((6 * CHUNK,), jnp.int32),
        pltpu.VMEM((6 * CHUNK,), jnp.float32),
        pltpu.VMEM((ZB,), jnp.float32),
        pltpu.VMEM_SHARED((ACCW,), jnp.float32),
        pltpu.VMEM_SHARED((NP,), jnp.float32),
        pltpu.SemaphoreType.DMA,
    ],
)
def _sc_scatter(ei, eax_h, eay_h, xop, xap, out, *scratch):
    _sc_body(ei, eax_h, eay_h, xop, xap, out, *scratch)


_KC = 20096


def _combine_body(a_ref, o_ref):
    a = a_ref[...]
    s = a[0:12] + a[12:24]
    mcxp = jnp.maximum(s[4:5], 1.0)
    mcxm = jnp.maximum(s[5:6], 1.0)
    mcyp = jnp.maximum(s[10:11], 1.0)
    mcym = jnp.maximum(s[11:12], 1.0)
    axp = s[0:1] / mcxp
    axm = s[1:2] / mcxm
    uxp = s[2:3] / mcxp
    uxm = s[3:4] / mcxm
    ayp = s[6:7] / mcyp
    aym = s[7:8] / mcym
    uyp = s[8:9] / mcyp
    uym = s[9:10] / mcym
    loss = (axp * uxp - axm * uxm) / DELTA_X \
         + (ayp * uyp - aym * uym) / DELTA_Y + F_CONST
    o_ref[...] = loss


_combine = pl.pallas_call(
    _combine_body,
    grid=(NP // _KC,),
    in_specs=[pl.BlockSpec((24, _KC), lambda i: (0, i))],
    out_specs=pl.BlockSpec((1, _KC), lambda i: (0, i)),
    out_shape=jax.ShapeDtypeStruct((1, NP), jnp.float32),
)


@jax.jit
def kernel(x_out, x_a, edge_attr, edge_index):
    xop = jnp.pad(x_out[:, 0], (0, NP - N))
    xap = x_a[:, 0]
    eax_h = edge_attr[:, 0]
    eay_h = edge_attr[:, 1]
    acc = _sc_scatter(edge_index, eax_h, eay_h, xop, xap)
    loss = _combine(acc.reshape(24, NP))
    return loss.reshape(NP, 1)[:N]


# speedup vs baseline: 250.0924x; 1.0490x over previous
import functools

import jax
import jax.numpy as jnp
from jax import lax
from jax.experimental import pallas as pl
from jax.experimental.pallas import tpu as pltpu
from jax.experimental.pallas import tpu_sc as plsc

N = 100000
E = 6400000
NP = 100480
DUMP = 100224
DELTA_X = 0.1
DELTA_Y = 0.1
F_CONST = 1.0

NC = 2
NS = 16
CHUNK = 2048
NCHUNKS = E // CHUNK
ACCW = 12 * NP
ZW = 76800
ZWL = ACCW - 15 * ZW
ZB = 1920
TS = 6400
TSL = NP - 15 * TS

_mesh = plsc.VectorSubcoreMesh(core_axis_name="c", subcore_axis_name="s")

_GDN = lax.GatherDimensionNumbers(
    offset_dims=(), collapsed_slice_dims=(0,), start_index_map=(0,))


def _dyn_gather(v, idx):
    return lax.gather(v, idx[:, None], _GDN, slice_sizes=(1,),
                      mode=lax.GatherScatterMode.PROMISE_IN_BOUNDS)


def _sc_body(ei, eax_h, eay_h, xop, xap, out,
             srcb, dstb, eaxb, eayb, xab, xosb, xodb, sidx, sval, zbuf,
             acc, xo_sh, sem):
    cid = lax.axis_index("c")
    sid = lax.axis_index("s")
    wid = cid * NS + sid

    iota = lax.broadcasted_iota(jnp.int32, (16,), 0)
    zeros16 = jnp.zeros((16,), jnp.float32)
    ones16 = jnp.ones((16,), jnp.float32)

    toff = sid * TS

    @pl.when(sid < 15)
    def _():
        pltpu.sync_copy(xop.at[pl.ds(toff, TS)], xo_sh.at[pl.ds(toff, TS)])

    @pl.when(sid == 15)
    def _():
        pltpu.sync_copy(xop.at[pl.ds(15 * TS, TSL)], xo_sh.at[pl.ds(15 * TS, TSL)])

    def zb_body(i, _):
        zbuf[pl.ds(i * 16, 16)] = zeros16
        return 0
    lax.fori_loop(0, ZB // 16, zb_body, 0)

    def zc_body(i, _):
        pltpu.sync_copy(zbuf, acc.at[pl.ds(sid * ZW + i * ZB, ZB)])
        return 0
    ncp = jnp.where(sid < 15, ZW // ZB, ZWL // ZB)
    lax.fori_loop(0, ncp, zc_body, 0)

    def pf_body(i, _):
        sval[pl.ds(2 * CHUNK + i * 16, 16)] = ones16
        sval[pl.ds(5 * CHUNK + i * 16, 16)] = ones16
        return 0
    lax.fori_loop(0, CHUNK // 16, pf_body, 0)

    plsc.subcore_barrier()

    nch = jnp.where(wid < NCHUNKS % 32, NCHUNKS // 32 + 1, NCHUNKS // 32)
    base = wid * (NCHUNKS // 32) + jnp.minimum(wid, NCHUNKS % 32)

    def fire(ch, off):
        pltpu.async_copy(ei.at[0, pl.ds(ch * CHUNK, CHUNK)],
                         srcb.at[pl.ds(off, CHUNK)], sem)
        pltpu.async_copy(ei.at[1, pl.ds(ch * CHUNK, CHUNK)],
                         dstb.at[pl.ds(off, CHUNK)], sem)
        pltpu.async_copy(eax_h.at[pl.ds(ch * CHUNK, CHUNK)],
                         eaxb.at[pl.ds(off, CHUNK)], sem)
        pltpu.async_copy(eay_h.at[pl.ds(ch * CHUNK, CHUNK)],
                         eayb.at[pl.ds(off, CHUNK)], sem)

    fire(base, 0)

    def chunk_body(i, _):
        off = lax.rem(i, 2) * CHUNK
        pltpu.make_async_copy(ei.at[0, pl.ds(0, CHUNK)],
                              srcb.at[pl.ds(off, CHUNK)], sem).wait()
        pltpu.make_async_copy(ei.at[1, pl.ds(0, CHUNK)],
                              dstb.at[pl.ds(off, CHUNK)], sem).wait()
        pltpu.make_async_copy(eax_h.at[pl.ds(0, CHUNK)],
                              eaxb.at[pl.ds(off, CHUNK)], sem).wait()
        pltpu.make_async_copy(eay_h.at[pl.ds(0, CHUNK)],
                              eayb.at[pl.ds(off, CHUNK)], sem).wait()

        @pl.when(i + 1 < nch)
        def _():
            fire(base + i + 1, CHUNK - off)

        srcc = srcb.at[pl.ds(off, CHUNK)]
        dstc = dstb.at[pl.ds(off, CHUNK)]
        pltpu.sync_copy(xap.at[srcc], xab)
        pltpu.sync_copy(xo_sh.at[srcc], xosb)
        pltpu.sync_copy(xo_sh.at[dstc], xodb)

        def step(t, _):
            l = t * 16
            dst16 = dstb[pl.ds(off + l, 16)]
            xa16 = xab[pl.ds(l, 16)]
            du = xodb[pl.ds(l, 16)] - xosb[pl.ds(l, 16)]
            eax = eaxb[pl.ds(off + l, 16)]
            eay = eayb[pl.ds(off + l, 16)]

            def onedir(ea, b_sa, b_su, b_c):
                neg = ea < 0.0
                m = ea != 0.0
                b0 = dst16 + jnp.where(neg, NP, 0)
                bs = jnp.where(m, b0, DUMP)
                return bs + b_sa, bs + b_su, bs + b_c, du / ea

            ix_sa, ix_su, ix_c, vx = onedir(eax, 0, 2 * NP, 4 * NP)
            iy_sa, iy_su, iy_c, vy = onedir(eay, 6 * NP, 8 * NP, 10 * NP)

            sidx[pl.ds(l, 16)] = ix_sa
            sidx[pl.ds(CHUNK + l, 16)] = ix_su
            sidx[pl.ds(2 * CHUNK + l, 16)] = ix_c
            sidx[pl.ds(3 * CHUNK + l, 16)] = iy_sa
            sidx[pl.ds(4 * CHUNK + l, 16)] = iy_su
            sidx[pl.ds(5 * CHUNK + l, 16)] = iy_c
            sval[pl.ds(l, 16)] = xa16
            sval[pl.ds(CHUNK + l, 16)] = vx
            sval[pl.ds(3 * CHUNK + l, 16)] = xa16
            sval[pl.ds(4 * CHUNK + l, 16)] = vy
            return 0

        lax.fori_loop(0, CHUNK // 16, step, 0)
        pltpu.sync_copy(sval, acc.at[sidx], add=True)
        return 0

    lax.fori_loop(0, nch, chunk_body, 0)

    plsc.subcore_barrier()

    obase = cid * ACCW + sid * ZW

    @pl.when(sid < 15)
    def _():
        pltpu.sync_copy(acc.at[pl.ds(sid * ZW, ZW)], out.at[pl.ds(obase, ZW)])

    @pl.when(sid == 15)
    def _():
        pltpu.sync_copy(acc.at[pl.ds(15 * ZW, ZWL)],
                        out.at[pl.ds(cid * ACCW + 15 * ZW, ZWL)])


@functools.partial(
    pl.kernel,
    out_type=jax.ShapeDtypeStruct((NC * ACCW,), jnp.float32),
    mesh=_mesh,
    scratch_types=[
        pltpu.VMEM((2 * CHUNK,), jnp.int32),
        pltpu.VMEM((2 * CHUNK,), jnp.int32),
        pltpu.VMEM((2 * CHUNK,), jnp.float32),
        pltpu.VMEM((2 * CHUNK,), jnp.float32),
        pltpu.VMEM((CHUNK,), jnp.float32),
        pltpu.VMEM((CHUNK,), jnp.float32),
        pltpu.VMEM((CHUNK,), jnp.float32),
        pltpu.VMEM((6 * CHUNK,), jnp.int32),
        pltpu.VMEM((6 * CHUNK,), jnp.float32),
        pltpu.VMEM((ZB,), jnp.float32),
        pltpu.VMEM_SHARED((ACCW,), jnp.float32),
        pltpu.VMEM_SHARED((NP,), jnp.float32),
        pltpu.SemaphoreType.DMA,
    ],
)
def _sc_scatter(ei, eax_h, eay_h, xop, xap, out, *scratch):
    _sc_body(ei, eax_h, eay_h, xop, xap, out, *scratch)


_KC = 20096


def _combine_body(a_ref, o_ref):
    a = a_ref[...]
    s = a[0:12] + a[12:24]
    mcxp = jnp.maximum(s[4:5], 1.0)
    mcxm = jnp.maximum(s[5:6], 1.0)
    mcyp = jnp.maximum(s[10:11], 1.0)
    mcym = jnp.maximum(s[11:12], 1.0)
    axp = s[0:1] / mcxp
    axm = s[1:2] / mcxm
    uxp = s[2:3] / mcxp
    uxm = s[3:4] / mcxm
    ayp = s[6:7] / mcyp
    aym = s[7:8] / mcym
    uyp = s[8:9] / mcyp
    uym = s[9:10] / mcym
    loss = (axp * uxp - axm * uxm) / DELTA_X \
         + (ayp * uyp - aym * uym) / DELTA_Y + F_CONST
    o_ref[...] = loss


_combine = pl.pallas_call(
    _combine_body,
    grid=(NP // _KC,),
    in_specs=[pl.BlockSpec((24, _KC), lambda i: (0, i))],
    out_specs=pl.BlockSpec((1, _KC), lambda i: (0, i)),
    out_shape=jax.ShapeDtypeStruct((1, NP), jnp.float32),
)


@jax.jit
def kernel(x_out, x_a, edge_attr, edge_index):
    xop = jnp.pad(x_out[:, 0], (0, NP - N))
    xap = x_a[:, 0]
    eax_h = edge_attr[:, 0]
    eay_h = edge_attr[:, 1]
    acc = _sc_scatter(edge_index, eax_h, eay_h, xop, xap)
    loss = _combine(acc.reshape(24, NP))
    return loss.reshape(NP, 1)[:N]
